# baseline (speedup 1.0000x reference)
import functools

import jax
import jax.numpy as jnp
from jax import lax
from jax.experimental import pallas as pl
from jax.experimental.pallas import tpu as pltpu
from jax.experimental.pallas import tpu_sc as plsc

NUM_WORDS = 10000
HIDDEN = 128
N_EDGES = 320000
N_DOCS = 20000
NNZ_X = 1000000

NCORE = 2
NSUB = 16
L = 16

KX = 128
KH2 = KX // 2
SUP = 16
KSUP = KX * SUP
NBUF = 2
NNZ_PAD = NNZ_X + KSUP
RPT = 624
ZBLK = 64

_mesh = plsc.VectorSubcoreMesh(core_axis_name="c", subcore_axis_name="s")


def _zero_acc(buf, acc, s):
    zv = jnp.zeros((L,), jnp.float32)

    def zrow(i, _):
        for h in range(HIDDEN // L):
            buf[i, pl.ds(h * L, L)] = zv
        return 0

    lax.fori_loop(0, ZBLK, zrow, 0)
    row0 = s * RPT
    for j in range(9):
        pltpu.sync_copy(buf.at[pl.ds(0, ZBLK)],
                        acc.at[pl.ds(row0 + j * ZBLK, ZBLK)])
    pltpu.sync_copy(buf.at[pl.ds(0, 48)], acc.at[pl.ds(row0 + 576, 48)])

    @pl.when(s == NSUB - 1)
    def _():
        pltpu.sync_copy(buf.at[pl.ds(0, 16)],
                        acc.at[pl.ds(NSUB * RPT, 16)])


def _scale_rows(rows_v, val_v, row0):
    splat_idx = [jnp.full((L, 1), i, jnp.int32) for i in range(L)]
    dnums = lax.GatherDimensionNumbers(
        offset_dims=(), collapsed_slice_dims=(0,), start_index_map=(0,))

    def body(g, _):
        vv = val_v[pl.ds(row0 + g * L, L)]
        for lidx in range(L):
            v = lax.gather(vv, splat_idx[lidx], dnums, slice_sizes=(1,),
                           mode=lax.GatherScatterMode.PROMISE_IN_BOUNDS)
            e = row0 + g * L + lidx
            for h in range(HIDDEN // L):
                sl = pl.ds(h * L, L)
                rows_v[e, sl] = rows_v[e, sl] * v
        return 0

    lax.fori_loop(0, KH2 // L, body, 0)


def _gss_body(sidx_hbm, gidx_hbm, xval_hbm, t_hbm, bnd_hbm, out_hbm,
              bnd_vm, sidx_v, gidx_v, xval_v, vsel_v, scat0, scat1,
              rows0, rows1, acc, gsem0, gsem1, ssem0, ssem1):
    c = lax.axis_index("c")
    s = lax.axis_index("s")

    _zero_acc(rows0, acc, s)

    pltpu.sync_copy(bnd_hbm, bnd_vm)
    bv = bnd_vm[...]
    is0 = c == 0
    lo_c = jnp.where(is0, bv[0], bv[2])
    hi_c = jnp.where(is0, bv[1], bv[3])
    mbase = jnp.where(is0, bv[4], bv[5])
    n_c = hi_c - lo_c
    p0 = lo_c + ((n_c * s) // NSUB // 8) * 8
    p1 = jnp.where(s == NSUB - 1, hi_c,
                   lo_c + ((n_c * (s + 1)) // NSUB // 8) * 8)
    nsup = (p1 - p0 + (KSUP - 1)) // KSUP

    plsc.subcore_barrier()

    lane = lax.iota(jnp.int32, L)
    rows = (rows0, rows1)
    scat = (scat0, scat1)
    gsem = (gsem0, gsem1)
    ssem = (ssem0, ssem1)

    def sup(i, _):
        base = pl.multiple_of(p0 + i * KSUP, 8)
        pltpu.sync_copy(sidx_hbm.at[pl.ds(base, KSUP)], sidx_v)
        pltpu.sync_copy(gidx_hbm.at[pl.ds(base, KSUP)], gidx_v)
        pltpu.sync_copy(xval_hbm.at[pl.ds(base, KSUP)], xval_v)
        desc_s = [[], []]
        pltpu.async_copy(t_hbm.at[gidx_v.at[pl.ds(0, KX)]], rows[0],
                         gsem[0])
        for j in range(SUP):
            b = j % NBUF
            nb = (j + 1) % NBUF
            pltpu.make_async_copy(
                t_hbm.at[gidx_v.at[pl.ds(j * KX, KX)]], rows[b],
                gsem[b]).wait()
            for g in range(KX // L):
                sl_src = pl.ds(j * KX + g * L, L)
                sl_dst = pl.ds((g % (KH2 // L)) * L, L)
                d = sidx_v[sl_src]
                v = xval_v[sl_src]
                pos = base + j * KX + g * L + lane
                ok = (pos < p1) & (d >= mbase) & (d < mbase + NUM_WORDS)
                vsel_v[pl.ds(g * L, L)] = jnp.where(ok, v,
                                                    jnp.float32(0.0))
                scat[b][g // (KH2 // L), sl_dst] = (
                    jnp.clip(d - mbase, 0, NUM_WORDS - 1))
            if j + 1 < SUP:
                for dd in desc_s[nb]:
                    dd.wait()
                desc_s[nb] = []
                pltpu.async_copy(
                    t_hbm.at[gidx_v.at[pl.ds((j + 1) * KX, KX)]],
                    rows[nb], gsem[nb])
            desc_s[b] = []
            for hh in range(2):
                _scale_rows(rows[b], vsel_v, hh * KH2)
                desc_s[b].append(pltpu.async_copy(
                    rows[b].at[pl.ds(hh * KH2, KH2)],
                    acc.at[scat[b].at[hh]], ssem[b], add=True))
        for b in range(NBUF):
            for dd in desc_s[b]:
                dd.wait()
        return 0

    lax.fori_loop(0, nsup, sup, 0)

    plsc.subcore_barrier()
    row0 = s * RPT
    dst = out_hbm.at[pl.ds(c * NUM_WORDS, NUM_WORDS)]
    pltpu.sync_copy(acc.at[pl.ds(row0, RPT)], dst.at[pl.ds(row0, RPT)])

    @pl.when(s == NSUB - 1)
    def _():
        pltpu.sync_copy(acc.at[pl.ds(NSUB * RPT, 16)],
                        dst.at[pl.ds(NSUB * RPT, 16)])


_gss_call = functools.partial(
    pl.kernel,
    _gss_body,
    out_type=jax.ShapeDtypeStruct((NCORE * NUM_WORDS, HIDDEN), jnp.float32),
    mesh=_mesh,
    compiler_params=pltpu.CompilerParams(use_tc_tiling_on_sc=False),
    scratch_types=[
        pltpu.VMEM((L,), jnp.int32),
        pltpu.VMEM((KSUP,), jnp.int32),
        pltpu.VMEM((KSUP,), jnp.int32),
        pltpu.VMEM((KSUP,), jnp.float32),
        pltpu.VMEM((KX,), jnp.float32),
        pltpu.VMEM((2, KH2), jnp.int32),
        pltpu.VMEM((2, KH2), jnp.int32),
        pltpu.VMEM((KX, HIDDEN), jnp.float32),
        pltpu.VMEM((KX, HIDDEN), jnp.float32),
        pltpu.VMEM_SHARED((NUM_WORDS, HIDDEN), jnp.float32),
        pltpu.SemaphoreType.DMA,
        pltpu.SemaphoreType.DMA,
        pltpu.SemaphoreType.DMA,
        pltpu.SemaphoreType.DMA,
    ],
)()


_EDGE_BOUNDS = (0, N_EDGES // NCORE, N_EDGES // NCORE, N_EDGES, 0, 0)


def _bounds_arr(vals):
    b = jnp.zeros((L,), jnp.int32)
    for i, v in enumerate(vals):
        b = b.at[i].set(v)
    return b


_MBLK = 1000


def _mm_relu_body(p_ref, w_ref, o_ref):
    x = p_ref[0] + p_ref[1]
    y = lax.dot_general(x, w_ref[...], (((1,), (1,)), ((), ())),
                        preferred_element_type=jnp.float32)
    o_ref[...] = jnp.maximum(y, 0.0)


def _mm_relu(p, w):
    return pl.pallas_call(
        _mm_relu_body,
        grid=(NUM_WORDS // _MBLK,),
        in_specs=[
            pl.BlockSpec((NCORE, _MBLK, HIDDEN), lambda i: (0, i, 0)),
            pl.BlockSpec((HIDDEN, HIDDEN), lambda i: (0, 0)),
        ],
        out_specs=pl.BlockSpec((_MBLK, HIDDEN), lambda i: (i, 0)),
        out_shape=jax.ShapeDtypeStruct((NUM_WORDS, HIDDEN), jnp.float32),
    )(p.reshape(NCORE, NUM_WORDS, HIDDEN), w)


def _mm3_ln_body(p_ref, w_ref, emb_ref, g_ref, b_ref, o_ref):
    x = p_ref[0] + p_ref[1]
    y = lax.dot_general(x, w_ref[...], (((1,), (1,)), ((), ())),
                        preferred_element_type=jnp.float32)
    y = jnp.maximum(y, 0.0)
    e = emb_ref[...]
    hh = jnp.float32(0.3) * e + jnp.float32(0.7) * y
    mean = jnp.mean(hh, axis=1, keepdims=True)
    var = jnp.mean((hh - mean) ** 2, axis=1, keepdims=True)
    wh = (hh - mean) / jnp.sqrt(var + jnp.float32(1e-5))
    wh = wh * g_ref[...] + b_ref[...]
    o_ref[...] = wh + e


def _mm3_ln(p, w, emb, gamma, beta):
    return pl.pallas_call(
        _mm3_ln_body,
        grid=(NUM_WORDS // _MBLK,),
        in_specs=[
            pl.BlockSpec((NCORE, _MBLK, HIDDEN), lambda i: (0, i, 0)),
            pl.BlockSpec((HIDDEN, HIDDEN), lambda i: (0, 0)),
            pl.BlockSpec((_MBLK, HIDDEN), lambda i: (i, 0)),
            pl.BlockSpec((1, HIDDEN), lambda i: (0, 0)),
            pl.BlockSpec((1, HIDDEN), lambda i: (0, 0)),
        ],
        out_specs=pl.BlockSpec((_MBLK, HIDDEN), lambda i: (i, 0)),
        out_shape=jax.ShapeDtypeStruct((NUM_WORDS, HIDDEN), jnp.float32),
    )(p.reshape(NCORE, NUM_WORDS, HIDDEN), w, emb, gamma.reshape(1, HIDDEN),
      beta.reshape(1, HIDDEN))


def _mlp_body(dh_ref, wm1_ref, bm1_ref, wm2_ref, bm2_ref, wc_ref, bc_ref,
              o_ref):
    x = dh_ref[...]
    h1 = lax.dot_general(x, wm1_ref[...], (((1,), (1,)), ((), ())),
                         preferred_element_type=jnp.float32)
    h1 = jnp.maximum(h1 + bm1_ref[...], 0.0)
    h2 = lax.dot_general(h1, wm2_ref[...], (((1,), (1,)), ((), ())),
                         preferred_element_type=jnp.float32)
    h2 = jnp.maximum(h2 + bm2_ref[...], 0.0)
    lg = lax.dot_general(h2, wc_ref[...], (((1,), (1,)), ((), ())),
                         preferred_element_type=jnp.float32)
    o_ref[...] = lg + bc_ref[...]


def _mlp(doc_h, wm1, bm1, wm2, bm2, wc, bc):
    hid2 = HIDDEN // 2
    wc_pad = jnp.zeros((8, hid2), jnp.float32).at[:2].set(wc)
    bc_pad = jnp.zeros((1, 8), jnp.float32).at[0, :2].set(bc)
    out = pl.pallas_call(
        _mlp_body,
        grid=(N_DOCS // _MBLK,),
        in_specs=[
            pl.BlockSpec((_MBLK, HIDDEN), lambda i: (i, 0)),
            pl.BlockSpec((HIDDEN, HIDDEN), lambda i: (0, 0)),
            pl.BlockSpec((1, HIDDEN), lambda i: (0, 0)),
            pl.BlockSpec((hid2, HIDDEN), lambda i: (0, 0)),
            pl.BlockSpec((1, hid2), lambda i: (0, 0)),
            pl.BlockSpec((8, hid2), lambda i: (0, 0)),
            pl.BlockSpec((1, 8), lambda i: (0, 0)),
        ],
        out_specs=pl.BlockSpec((_MBLK, 8), lambda i: (i, 0)),
        out_shape=jax.ShapeDtypeStruct((N_DOCS, 8), jnp.float32),
    )(doc_h, wm1, bm1.reshape(1, HIDDEN), wm2, bm2.reshape(1, hid2),
      wc_pad, bc_pad)
    return lax.slice(out, (0, 0), (N_DOCS, 2))


def kernel(A_indices, A_values, X_doc_idx, X_word_idx, X_values, emb,
           W1, W2, W3, ln_gamma, ln_beta, Wm1, bm1, Wm2, bm2, Wc, bc):
    epad = NNZ_PAD - N_EDGES
    row = jnp.pad(A_indices[0].astype(jnp.int32), (0, epad))
    col = jnp.pad(A_indices[1].astype(jnp.int32), (0, epad))
    aval = jnp.pad(A_values, (0, epad))
    ebnd = _bounds_arr(_EDGE_BOUNDS)

    p = _gss_call(row, col, aval, emb, ebnd)
    h = _mm_relu(p, W1)
    p = _gss_call(row, col, aval, h, ebnd)
    h = _mm_relu(p, W2)
    p = _gss_call(row, col, aval, h, ebnd)
    t = _mm3_ln(p, W3, emb, ln_gamma, ln_beta)

    didx = X_doc_idx.astype(jnp.int32)
    widx = X_word_idx.astype(jnp.int32)
    xpad = NNZ_PAD - NNZ_X
    didx_p = jnp.pad(didx, (0, xpad), constant_values=N_DOCS - 1)
    widx_p = jnp.pad(widx, (0, xpad))
    xval_p = jnp.pad(X_values, (0, xpad))

    split = jnp.searchsorted(didx, N_DOCS // NCORE).astype(jnp.int32)
    hi0 = ((split + 7) // 8) * 8
    lo1 = (split // 8) * 8
    xbnd = _bounds_arr((0, hi0, lo1, NNZ_X, 0, NUM_WORDS))

    doc_h = _gss_call(didx_p, widx_p, xval_p, t, xbnd)
    return _mlp(doc_h, Wm1, bm1, Wm2, bm2, Wc, bc)


# speedup vs baseline: 1.0116x; 1.0116x over previous
import functools

import jax
import jax.numpy as jnp
from jax import lax
from jax.experimental import pallas as pl
from jax.experimental.pallas import tpu as pltpu
from jax.experimental.pallas import tpu_sc as plsc

NUM_WORDS = 10000
HIDDEN = 128
N_EDGES = 320000
N_DOCS = 20000
NNZ_X = 1000000

NCORE = 2
NSUB = 16
L = 16

KX = 128
KH2 = KX // 2
SUP = 8
KSUP = KX * SUP
NBUF = 2
NNZ_PAD = NNZ_X + KSUP
RPT = 624
ZBLK = 64

_mesh = plsc.VectorSubcoreMesh(core_axis_name="c", subcore_axis_name="s")


def _zero_acc(buf, acc, s):
    zv = jnp.zeros((L,), jnp.float32)

    def zrow(i, _):
        for h in range(HIDDEN // L):
            buf[i, pl.ds(h * L, L)] = zv
        return 0

    lax.fori_loop(0, ZBLK, zrow, 0)
    row0 = s * RPT
    for j in range(9):
        pltpu.sync_copy(buf.at[pl.ds(0, ZBLK)],
                        acc.at[pl.ds(row0 + j * ZBLK, ZBLK)])
    pltpu.sync_copy(buf.at[pl.ds(0, 48)], acc.at[pl.ds(row0 + 576, 48)])

    @pl.when(s == NSUB - 1)
    def _():
        pltpu.sync_copy(buf.at[pl.ds(0, 16)],
                        acc.at[pl.ds(NSUB * RPT, 16)])


def _scale_rows(rows_v, val_v, row0):
    splat_idx = [jnp.full((L, 1), i, jnp.int32) for i in range(L)]
    dnums = lax.GatherDimensionNumbers(
        offset_dims=(), collapsed_slice_dims=(0,), start_index_map=(0,))

    def body(g, _):
        vv = val_v[pl.ds(row0 + g * L, L)]
        for lidx in range(L):
            v = lax.gather(vv, splat_idx[lidx], dnums, slice_sizes=(1,),
                           mode=lax.GatherScatterMode.PROMISE_IN_BOUNDS)
            e = row0 + g * L + lidx
            for h in range(HIDDEN // L):
                sl = pl.ds(h * L, L)
                rows_v[e, sl] = rows_v[e, sl] * v
        return 0

    lax.fori_loop(0, KH2 // L, body, 0)


def _gss_body(sidx_hbm, gidx_hbm, xval_hbm, t_hbm, bnd_hbm, out_hbm,
              bnd_vm, sidx_v, gidx_v, xval_v, vsel_v, scat0, scat1,
              rows0, rows1, acc, gsem0, gsem1, ssem0, ssem1):
    c = lax.axis_index("c")
    s = lax.axis_index("s")

    _zero_acc(rows0, acc, s)

    pltpu.sync_copy(bnd_hbm, bnd_vm)
    bv = bnd_vm[...]
    is0 = c == 0
    lo_c = jnp.where(is0, bv[0], bv[2])
    hi_c = jnp.where(is0, bv[1], bv[3])
    mbase = jnp.where(is0, bv[4], bv[5])
    n_c = hi_c - lo_c
    p0 = lo_c + ((n_c * s) // NSUB // 8) * 8
    p1 = jnp.where(s == NSUB - 1, hi_c,
                   lo_c + ((n_c * (s + 1)) // NSUB // 8) * 8)
    nsup = (p1 - p0 + (KSUP - 1)) // KSUP

    plsc.subcore_barrier()

    lane = lax.iota(jnp.int32, L)
    rows = (rows0, rows1)
    scat = (scat0, scat1)
    gsem = (gsem0, gsem1)
    ssem = (ssem0, ssem1)

    def sup(i, _):
        base = pl.multiple_of(p0 + i * KSUP, 8)
        pltpu.sync_copy(sidx_hbm.at[pl.ds(base, KSUP)], sidx_v)
        pltpu.sync_copy(gidx_hbm.at[pl.ds(base, KSUP)], gidx_v)
        pltpu.sync_copy(xval_hbm.at[pl.ds(base, KSUP)], xval_v)
        desc_s = [[], []]
        pltpu.async_copy(t_hbm.at[gidx_v.at[pl.ds(0, KX)]], rows[0],
                         gsem[0])
        for j in range(SUP):
            b = j % NBUF
            nb = (j + 1) % NBUF
            pltpu.make_async_copy(
                t_hbm.at[gidx_v.at[pl.ds(j * KX, KX)]], rows[b],
                gsem[b]).wait()
            for g in range(KX // L):
                sl_src = pl.ds(j * KX + g * L, L)
                sl_dst = pl.ds((g % (KH2 // L)) * L, L)
                d = sidx_v[sl_src]
                v = xval_v[sl_src]
                pos = base + j * KX + g * L + lane
                ok = (pos < p1) & (d >= mbase) & (d < mbase + NUM_WORDS)
                vsel_v[pl.ds(g * L, L)] = jnp.where(ok, v,
                                                    jnp.float32(0.0))
                scat[b][g // (KH2 // L), sl_dst] = (
                    jnp.clip(d - mbase, 0, NUM_WORDS - 1))
            if j + 1 < SUP:
                for dd in desc_s[nb]:
                    dd.wait()
                desc_s[nb] = []
                pltpu.async_copy(
                    t_hbm.at[gidx_v.at[pl.ds((j + 1) * KX, KX)]],
                    rows[nb], gsem[nb])
            desc_s[b] = []
            for hh in range(2):
                _scale_rows(rows[b], vsel_v, hh * KH2)
                desc_s[b].append(pltpu.async_copy(
                    rows[b].at[pl.ds(hh * KH2, KH2)],
                    acc.at[scat[b].at[hh]], ssem[b], add=True))
        for b in range(NBUF):
            for dd in desc_s[b]:
                dd.wait()
        return 0

    lax.fori_loop(0, nsup, sup, 0)

    plsc.subcore_barrier()
    row0 = s * RPT
    dst = out_hbm.at[pl.ds(c * NUM_WORDS, NUM_WORDS)]
    pltpu.sync_copy(acc.at[pl.ds(row0, RPT)], dst.at[pl.ds(row0, RPT)])

    @pl.when(s == NSUB - 1)
    def _():
        pltpu.sync_copy(acc.at[pl.ds(NSUB * RPT, 16)],
                        dst.at[pl.ds(NSUB * RPT, 16)])


_gss_call = functools.partial(
    pl.kernel,
    _gss_body,
    out_type=jax.ShapeDtypeStruct((NCORE * NUM_WORDS, HIDDEN), jnp.float32),
    mesh=_mesh,
    scratch_types=[
        pltpu.VMEM((L,), jnp.int32),
        pltpu.VMEM((KSUP,), jnp.int32),
        pltpu.VMEM((KSUP,), jnp.int32),
        pltpu.VMEM((KSUP,), jnp.float32),
        pltpu.VMEM((KX,), jnp.float32),
        pltpu.VMEM((2, KH2), jnp.int32),
        pltpu.VMEM((2, KH2), jnp.int32),
        pltpu.VMEM((KX, HIDDEN), jnp.float32),
        pltpu.VMEM((KX, HIDDEN), jnp.float32),
        pltpu.VMEM_SHARED((NUM_WORDS, HIDDEN), jnp.float32),
        pltpu.SemaphoreType.DMA,
        pltpu.SemaphoreType.DMA,
        pltpu.SemaphoreType.DMA,
        pltpu.SemaphoreType.DMA,
    ],
)()


_EDGE_BOUNDS = (0, N_EDGES // NCORE, N_EDGES // NCORE, N_EDGES, 0, 0)


def _bounds_arr(vals):
    b = jnp.zeros((L,), jnp.int32)
    for i, v in enumerate(vals):
        b = b.at[i].set(v)
    return b


_MBLK = 1000


def _mm_relu_body(p_ref, w_ref, o_ref):
    x = p_ref[0] + p_ref[1]
    y = lax.dot_general(x, w_ref[...], (((1,), (1,)), ((), ())),
                        preferred_element_type=jnp.float32)
    o_ref[...] = jnp.maximum(y, 0.0)


def _mm_relu(p, w):
    return pl.pallas_call(
        _mm_relu_body,
        grid=(NUM_WORDS // _MBLK,),
        in_specs=[
            pl.BlockSpec((NCORE, _MBLK, HIDDEN), lambda i: (0, i, 0)),
            pl.BlockSpec((HIDDEN, HIDDEN), lambda i: (0, 0)),
        ],
        out_specs=pl.BlockSpec((_MBLK, HIDDEN), lambda i: (i, 0)),
        out_shape=jax.ShapeDtypeStruct((NUM_WORDS, HIDDEN), jnp.float32),
    )(p.reshape(NCORE, NUM_WORDS, HIDDEN), w)


def _mm3_ln_body(p_ref, w_ref, emb_ref, g_ref, b_ref, o_ref):
    x = p_ref[0] + p_ref[1]
    y = lax.dot_general(x, w_ref[...], (((1,), (1,)), ((), ())),
                        preferred_element_type=jnp.float32)
    y = jnp.maximum(y, 0.0)
    e = emb_ref[...]
    hh = jnp.float32(0.3) * e + jnp.float32(0.7) * y
    mean = jnp.mean(hh, axis=1, keepdims=True)
    var = jnp.mean((hh - mean) ** 2, axis=1, keepdims=True)
    wh = (hh - mean) / jnp.sqrt(var + jnp.float32(1e-5))
    wh = wh * g_ref[...] + b_ref[...]
    o_ref[...] = wh + e


def _mm3_ln(p, w, emb, gamma, beta):
    return pl.pallas_call(
        _mm3_ln_body,
        grid=(NUM_WORDS // _MBLK,),
        in_specs=[
            pl.BlockSpec((NCORE, _MBLK, HIDDEN), lambda i: (0, i, 0)),
            pl.BlockSpec((HIDDEN, HIDDEN), lambda i: (0, 0)),
            pl.BlockSpec((_MBLK, HIDDEN), lambda i: (i, 0)),
            pl.BlockSpec((1, HIDDEN), lambda i: (0, 0)),
            pl.BlockSpec((1, HIDDEN), lambda i: (0, 0)),
        ],
        out_specs=pl.BlockSpec((_MBLK, HIDDEN), lambda i: (i, 0)),
        out_shape=jax.ShapeDtypeStruct((NUM_WORDS, HIDDEN), jnp.float32),
    )(p.reshape(NCORE, NUM_WORDS, HIDDEN), w, emb, gamma.reshape(1, HIDDEN),
      beta.reshape(1, HIDDEN))


def _mlp_body(dh_ref, wm1_ref, bm1_ref, wm2_ref, bm2_ref, wc_ref, bc_ref,
              o_ref):
    x = dh_ref[...]
    h1 = lax.dot_general(x, wm1_ref[...], (((1,), (1,)), ((), ())),
                         preferred_element_type=jnp.float32)
    h1 = jnp.maximum(h1 + bm1_ref[...], 0.0)
    h2 = lax.dot_general(h1, wm2_ref[...], (((1,), (1,)), ((), ())),
                         preferred_element_type=jnp.float32)
    h2 = jnp.maximum(h2 + bm2_ref[...], 0.0)
    lg = lax.dot_general(h2, wc_ref[...], (((1,), (1,)), ((), ())),
                         preferred_element_type=jnp.float32)
    o_ref[...] = lg + bc_ref[...]


def _mlp(doc_h, wm1, bm1, wm2, bm2, wc, bc):
    hid2 = HIDDEN // 2
    wc_pad = jnp.zeros((8, hid2), jnp.float32).at[:2].set(wc)
    bc_pad = jnp.zeros((1, 8), jnp.float32).at[0, :2].set(bc)
    out = pl.pallas_call(
        _mlp_body,
        grid=(N_DOCS // _MBLK,),
        in_specs=[
            pl.BlockSpec((_MBLK, HIDDEN), lambda i: (i, 0)),
            pl.BlockSpec((HIDDEN, HIDDEN), lambda i: (0, 0)),
            pl.BlockSpec((1, HIDDEN), lambda i: (0, 0)),
            pl.BlockSpec((hid2, HIDDEN), lambda i: (0, 0)),
            pl.BlockSpec((1, hid2), lambda i: (0, 0)),
            pl.BlockSpec((8, hid2), lambda i: (0, 0)),
            pl.BlockSpec((1, 8), lambda i: (0, 0)),
        ],
        out_specs=pl.BlockSpec((_MBLK, 8), lambda i: (i, 0)),
        out_shape=jax.ShapeDtypeStruct((N_DOCS, 8), jnp.float32),
    )(doc_h, wm1, bm1.reshape(1, HIDDEN), wm2, bm2.reshape(1, hid2),
      wc_pad, bc_pad)
    return lax.slice(out, (0, 0), (N_DOCS, 2))


def kernel(A_indices, A_values, X_doc_idx, X_word_idx, X_values, emb,
           W1, W2, W3, ln_gamma, ln_beta, Wm1, bm1, Wm2, bm2, Wc, bc):
    epad = NNZ_PAD - N_EDGES
    row = jnp.pad(A_indices[0].astype(jnp.int32), (0, epad))
    col = jnp.pad(A_indices[1].astype(jnp.int32), (0, epad))
    aval = jnp.pad(A_values, (0, epad))
    ebnd = _bounds_arr(_EDGE_BOUNDS)

    p = _gss_call(row, col, aval, emb, ebnd)
    h = _mm_relu(p, W1)
    p = _gss_call(row, col, aval, h, ebnd)
    h = _mm_relu(p, W2)
    p = _gss_call(row, col, aval, h, ebnd)
    t = _mm3_ln(p, W3, emb, ln_gamma, ln_beta)

    didx = X_doc_idx.astype(jnp.int32)
    widx = X_word_idx.astype(jnp.int32)
    xpad = NNZ_PAD - NNZ_X
    didx_p = jnp.pad(didx, (0, xpad), constant_values=N_DOCS - 1)
    widx_p = jnp.pad(widx, (0, xpad))
    xval_p = jnp.pad(X_values, (0, xpad))

    split = jnp.searchsorted(didx, N_DOCS // NCORE).astype(jnp.int32)
    hi0 = ((split + 7) // 8) * 8
    lo1 = (split // 8) * 8
    xbnd = _bounds_arr((0, hi0, lo1, NNZ_X, 0, NUM_WORDS))

    doc_h = _gss_call(didx_p, widx_p, xval_p, t, xbnd)
    return _mlp(doc_h, Wm1, bm1, Wm2, bm2, Wc, bc)


# speedup vs baseline: 1.0162x; 1.0045x over previous
import functools

import jax
import jax.numpy as jnp
from jax import lax
from jax.experimental import pallas as pl
from jax.experimental.pallas import tpu as pltpu
from jax.experimental.pallas import tpu_sc as plsc

NUM_WORDS = 10000
HIDDEN = 128
N_EDGES = 320000
N_DOCS = 20000
NNZ_X = 1000000

NCORE = 2
NSUB = 16
L = 16

KX = 128
KH2 = KX // 2
SUP = 8
KSUP = KX * SUP
NBUF = 2
NNZ_PAD = NNZ_X + KSUP
RPT = 624
ZBLK = 64

_mesh = plsc.VectorSubcoreMesh(core_axis_name="c", subcore_axis_name="s")


def _zero_acc(buf, acc, s):
    zv = jnp.zeros((L,), jnp.float32)

    def zrow(i, _):
        for h in range(HIDDEN // L):
            buf[i, pl.ds(h * L, L)] = zv
        return 0

    lax.fori_loop(0, ZBLK, zrow, 0)
    row0 = s * RPT
    for j in range(9):
        pltpu.sync_copy(buf.at[pl.ds(0, ZBLK)],
                        acc.at[pl.ds(row0 + j * ZBLK, ZBLK)])
    pltpu.sync_copy(buf.at[pl.ds(0, 48)], acc.at[pl.ds(row0 + 576, 48)])

    @pl.when(s == NSUB - 1)
    def _():
        pltpu.sync_copy(buf.at[pl.ds(0, 16)],
                        acc.at[pl.ds(NSUB * RPT, 16)])


def _scale_rows(rows_v, val_v, row0):
    splat_idx = [jnp.full((L, 1), i, jnp.int32) for i in range(L)]
    dnums = lax.GatherDimensionNumbers(
        offset_dims=(), collapsed_slice_dims=(0,), start_index_map=(0,))

    def body(g, _):
        vv = val_v[pl.ds(row0 + g * L, L)]
        for lidx in range(L):
            v = lax.gather(vv, splat_idx[lidx], dnums, slice_sizes=(1,),
                           mode=lax.GatherScatterMode.PROMISE_IN_BOUNDS)
            e = row0 + g * L + lidx
            for h in range(HIDDEN // L):
                sl = pl.ds(h * L, L)
                rows_v[e, sl] = rows_v[e, sl] * v
        return 0

    lax.fori_loop(0, KH2 // L, body, 0)


def _gss_body(sidx_hbm, gidx_hbm, xval_hbm, t_hbm, bnd_hbm, out_hbm,
              bnd_vm, sidx_v, gidx_v, xval_v, vsel_v, scat0, scat1,
              rows0, rows1, acc, gsem0, gsem1, ssem0, ssem1):
    c = lax.axis_index("c")
    s = lax.axis_index("s")

    _zero_acc(rows0, acc, s)

    pltpu.sync_copy(bnd_hbm, bnd_vm)
    bv = bnd_vm[...]
    is0 = c == 0
    lo_c = jnp.where(is0, bv[0], bv[2])
    hi_c = jnp.where(is0, bv[1], bv[3])
    mbase = jnp.where(is0, bv[4], bv[5])
    n_c = hi_c - lo_c
    p0 = lo_c + ((n_c * s) // NSUB // 8) * 8
    p1 = jnp.where(s == NSUB - 1, hi_c,
                   lo_c + ((n_c * (s + 1)) // NSUB // 8) * 8)
    nsup = (p1 - p0 + (KSUP - 1)) // KSUP

    plsc.subcore_barrier()

    lane = lax.iota(jnp.int32, L)
    rows = (rows0, rows1)
    scat = (scat0, scat1)
    gsem = (gsem0, gsem1)
    ssem = (ssem0, ssem1)

    def sup(i, _):
        base = pl.multiple_of(p0 + i * KSUP, 8)
        pltpu.sync_copy(sidx_hbm.at[pl.ds(base, KSUP)], sidx_v)
        pltpu.sync_copy(gidx_hbm.at[pl.ds(base, KSUP)], gidx_v)
        pltpu.sync_copy(xval_hbm.at[pl.ds(base, KSUP)], xval_v)
        desc_s = [[], []]
        pltpu.async_copy(t_hbm.at[gidx_v.at[pl.ds(0, KX)]], rows[0],
                         gsem[0])
        for j in range(SUP):
            b = j % NBUF
            nb = (j + 1) % NBUF
            pltpu.make_async_copy(
                t_hbm.at[gidx_v.at[pl.ds(j * KX, KX)]], rows[b],
                gsem[b]).wait()
            for g in range(KX // L):
                sl_src = pl.ds(j * KX + g * L, L)
                sl_dst = pl.ds((g % (KH2 // L)) * L, L)
                d = sidx_v[sl_src]
                v = xval_v[sl_src]
                pos = base + j * KX + g * L + lane
                ok = (pos < p1) & (d >= mbase) & (d < mbase + NUM_WORDS)
                vsel_v[pl.ds(g * L, L)] = jnp.where(ok, v,
                                                    jnp.float32(0.0))
                scat[b][g // (KH2 // L), sl_dst] = (
                    jnp.clip(d - mbase, 0, NUM_WORDS - 1))
            if j + 1 < SUP:
                for dd in desc_s[nb]:
                    dd.wait()
                desc_s[nb] = []
                pltpu.async_copy(
                    t_hbm.at[gidx_v.at[pl.ds((j + 1) * KX, KX)]],
                    rows[nb], gsem[nb])
            desc_s[b] = []
            for hh in range(2):
                _scale_rows(rows[b], vsel_v, hh * KH2)
                desc_s[b].append(pltpu.async_copy(
                    rows[b].at[pl.ds(hh * KH2, KH2)],
                    acc.at[scat[b].at[hh]], ssem[b], add=True))
        for b in range(NBUF):
            for dd in desc_s[b]:
                dd.wait()
        return 0

    lax.fori_loop(0, nsup, sup, 0)

    plsc.subcore_barrier()
    row0 = s * RPT
    dst = out_hbm.at[pl.ds(c * NUM_WORDS, NUM_WORDS)]
    pltpu.sync_copy(acc.at[pl.ds(row0, RPT)], dst.at[pl.ds(row0, RPT)])

    @pl.when(s == NSUB - 1)
    def _():
        pltpu.sync_copy(acc.at[pl.ds(NSUB * RPT, 16)],
                        dst.at[pl.ds(NSUB * RPT, 16)])


_gss_call = functools.partial(
    pl.kernel,
    _gss_body,
    out_type=jax.ShapeDtypeStruct((NCORE * NUM_WORDS, HIDDEN), jnp.float32),
    mesh=_mesh,
    compiler_params=pltpu.CompilerParams(use_tc_tiling_on_sc=False),
    scratch_types=[
        pltpu.VMEM((L,), jnp.int32),
        pltpu.VMEM((KSUP,), jnp.int32),
        pltpu.VMEM((KSUP,), jnp.int32),
        pltpu.VMEM((KSUP,), jnp.float32),
        pltpu.VMEM((KX,), jnp.float32),
        pltpu.VMEM((2, KH2), jnp.int32),
        pltpu.VMEM((2, KH2), jnp.int32),
        pltpu.VMEM((KX, HIDDEN), jnp.float32),
        pltpu.VMEM((KX, HIDDEN), jnp.float32),
        pltpu.VMEM_SHARED((NUM_WORDS, HIDDEN), jnp.float32),
        pltpu.SemaphoreType.DMA,
        pltpu.SemaphoreType.DMA,
        pltpu.SemaphoreType.DMA,
        pltpu.SemaphoreType.DMA,
    ],
)()


_EDGE_BOUNDS = (0, N_EDGES // NCORE, N_EDGES // NCORE, N_EDGES, 0, 0)


def _bounds_arr(vals):
    b = jnp.zeros((L,), jnp.int32)
    for i, v in enumerate(vals):
        b = b.at[i].set(v)
    return b


_MBLK = 1000


def _mm_relu_body(p_ref, w_ref, o_ref):
    x = p_ref[0] + p_ref[1]
    y = lax.dot_general(x, w_ref[...], (((1,), (1,)), ((), ())),
                        preferred_element_type=jnp.float32)
    o_ref[...] = jnp.maximum(y, 0.0)


def _mm_relu(p, w):
    return pl.pallas_call(
        _mm_relu_body,
        grid=(NUM_WORDS // _MBLK,),
        in_specs=[
            pl.BlockSpec((NCORE, _MBLK, HIDDEN), lambda i: (0, i, 0)),
            pl.BlockSpec((HIDDEN, HIDDEN), lambda i: (0, 0)),
        ],
        out_specs=pl.BlockSpec((_MBLK, HIDDEN), lambda i: (i, 0)),
        out_shape=jax.ShapeDtypeStruct((NUM_WORDS, HIDDEN), jnp.float32),
    )(p.reshape(NCORE, NUM_WORDS, HIDDEN), w)


def _mm3_ln_body(p_ref, w_ref, emb_ref, g_ref, b_ref, o_ref):
    x = p_ref[0] + p_ref[1]
    y = lax.dot_general(x, w_ref[...], (((1,), (1,)), ((), ())),
                        preferred_element_type=jnp.float32)
    y = jnp.maximum(y, 0.0)
    e = emb_ref[...]
    hh = jnp.float32(0.3) * e + jnp.float32(0.7) * y
    mean = jnp.mean(hh, axis=1, keepdims=True)
    var = jnp.mean((hh - mean) ** 2, axis=1, keepdims=True)
    wh = (hh - mean) / jnp.sqrt(var + jnp.float32(1e-5))
    wh = wh * g_ref[...] + b_ref[...]
    o_ref[...] = wh + e


def _mm3_ln(p, w, emb, gamma, beta):
    return pl.pallas_call(
        _mm3_ln_body,
        grid=(NUM_WORDS // _MBLK,),
        in_specs=[
            pl.BlockSpec((NCORE, _MBLK, HIDDEN), lambda i: (0, i, 0)),
            pl.BlockSpec((HIDDEN, HIDDEN), lambda i: (0, 0)),
            pl.BlockSpec((_MBLK, HIDDEN), lambda i: (i, 0)),
            pl.BlockSpec((1, HIDDEN), lambda i: (0, 0)),
            pl.BlockSpec((1, HIDDEN), lambda i: (0, 0)),
        ],
        out_specs=pl.BlockSpec((_MBLK, HIDDEN), lambda i: (i, 0)),
        out_shape=jax.ShapeDtypeStruct((NUM_WORDS, HIDDEN), jnp.float32),
    )(p.reshape(NCORE, NUM_WORDS, HIDDEN), w, emb, gamma.reshape(1, HIDDEN),
      beta.reshape(1, HIDDEN))


def _mlp_body(dh_ref, wm1_ref, bm1_ref, wm2_ref, bm2_ref, wc_ref, bc_ref,
              o_ref):
    x = dh_ref[...]
    h1 = lax.dot_general(x, wm1_ref[...], (((1,), (1,)), ((), ())),
                         preferred_element_type=jnp.float32)
    h1 = jnp.maximum(h1 + bm1_ref[...], 0.0)
    h2 = lax.dot_general(h1, wm2_ref[...], (((1,), (1,)), ((), ())),
                         preferred_element_type=jnp.float32)
    h2 = jnp.maximum(h2 + bm2_ref[...], 0.0)
    lg = lax.dot_general(h2, wc_ref[...], (((1,), (1,)), ((), ())),
                         preferred_element_type=jnp.float32)
    o_ref[...] = lg + bc_ref[...]


def _mlp(doc_h, wm1, bm1, wm2, bm2, wc, bc):
    hid2 = HIDDEN // 2
    wc_pad = jnp.zeros((8, hid2), jnp.float32).at[:2].set(wc)
    bc_pad = jnp.zeros((1, 8), jnp.float32).at[0, :2].set(bc)
    out = pl.pallas_call(
        _mlp_body,
        grid=(N_DOCS // _MBLK,),
        in_specs=[
            pl.BlockSpec((_MBLK, HIDDEN), lambda i: (i, 0)),
            pl.BlockSpec((HIDDEN, HIDDEN), lambda i: (0, 0)),
            pl.BlockSpec((1, HIDDEN), lambda i: (0, 0)),
            pl.BlockSpec((hid2, HIDDEN), lambda i: (0, 0)),
            pl.BlockSpec((1, hid2), lambda i: (0, 0)),
            pl.BlockSpec((8, hid2), lambda i: (0, 0)),
            pl.BlockSpec((1, 8), lambda i: (0, 0)),
        ],
        out_specs=pl.BlockSpec((_MBLK, 8), lambda i: (i, 0)),
        out_shape=jax.ShapeDtypeStruct((N_DOCS, 8), jnp.float32),
    )(doc_h, wm1, bm1.reshape(1, HIDDEN), wm2, bm2.reshape(1, hid2),
      wc_pad, bc_pad)
    return lax.slice(out, (0, 0), (N_DOCS, 2))


def kernel(A_indices, A_values, X_doc_idx, X_word_idx, X_values, emb,
           W1, W2, W3, ln_gamma, ln_beta, Wm1, bm1, Wm2, bm2, Wc, bc):
    epad = NNZ_PAD - N_EDGES
    row = jnp.pad(A_indices[0].astype(jnp.int32), (0, epad))
    col = jnp.pad(A_indices[1].astype(jnp.int32), (0, epad))
    aval = jnp.pad(A_values, (0, epad))
    ebnd = _bounds_arr(_EDGE_BOUNDS)

    p = _gss_call(row, col, aval, emb, ebnd)
    h = _mm_relu(p, W1)
    p = _gss_call(row, col, aval, h, ebnd)
    h = _mm_relu(p, W2)
    p = _gss_call(row, col, aval, h, ebnd)
    t = _mm3_ln(p, W3, emb, ln_gamma, ln_beta)

    didx = X_doc_idx.astype(jnp.int32)
    widx = X_word_idx.astype(jnp.int32)
    xpad = NNZ_PAD - NNZ_X
    didx_p = jnp.pad(didx, (0, xpad), constant_values=N_DOCS - 1)
    widx_p = jnp.pad(widx, (0, xpad))
    xval_p = jnp.pad(X_values, (0, xpad))

    split = jnp.searchsorted(didx, N_DOCS // NCORE).astype(jnp.int32)
    hi0 = ((split + 7) // 8) * 8
    lo1 = (split // 8) * 8
    xbnd = _bounds_arr((0, hi0, lo1, NNZ_X, 0, NUM_WORDS))

    doc_h = _gss_call(didx_p, widx_p, xval_p, t, xbnd)
    return _mlp(doc_h, Wm1, bm1, Wm2, bm2, Wc, bc)


# speedup vs baseline: 1.0622x; 1.0453x over previous
import functools

import jax
import jax.numpy as jnp
from jax import lax
from jax.experimental import pallas as pl
from jax.experimental.pallas import tpu as pltpu
from jax.experimental.pallas import tpu_sc as plsc

NUM_WORDS = 10000
HIDDEN = 128
N_EDGES = 320000
N_DOCS = 20000
NNZ_X = 1000000

NCORE = 2
NSUB = 16
L = 16

KX = 128
KH2 = KX // 2
SUP = 8
KSUP = KX * SUP
NBUF = 2
NNZ_PAD = NNZ_X + KSUP
RPT = 624
ZBLK = 64

_mesh = plsc.VectorSubcoreMesh(core_axis_name="c", subcore_axis_name="s")


def _zero_acc(buf, acc, s):
    zv = jnp.zeros((L,), jnp.float32)

    def zrow(i, _):
        for h in range(HIDDEN // L):
            buf[i, pl.ds(h * L, L)] = zv
        return 0

    lax.fori_loop(0, ZBLK, zrow, 0)
    row0 = s * RPT
    for j in range(9):
        pltpu.sync_copy(buf.at[pl.ds(0, ZBLK)],
                        acc.at[pl.ds(row0 + j * ZBLK, ZBLK)])
    pltpu.sync_copy(buf.at[pl.ds(0, 48)], acc.at[pl.ds(row0 + 576, 48)])

    @pl.when(s == NSUB - 1)
    def _():
        pltpu.sync_copy(buf.at[pl.ds(0, 16)],
                        acc.at[pl.ds(NSUB * RPT, 16)])


def _scale_rows(rows_v, val_v, row0):
    splat_idx = [jnp.full((L, 1), i, jnp.int32) for i in range(L)]
    dnums = lax.GatherDimensionNumbers(
        offset_dims=(), collapsed_slice_dims=(0,), start_index_map=(0,))

    def body(g, _):
        vv = val_v[pl.ds(row0 + g * L, L)]
        for lidx in range(L):
            v = lax.gather(vv, splat_idx[lidx], dnums, slice_sizes=(1,),
                           mode=lax.GatherScatterMode.PROMISE_IN_BOUNDS)
            e = row0 + g * L + lidx
            for h in range(HIDDEN // L):
                sl = pl.ds(h * L, L)
                rows_v[e, sl] = rows_v[e, sl] * v
        return 0

    lax.fori_loop(0, KH2 // L, body, 0)


def _gss_body(sidx_hbm, gidx_hbm, xval_hbm, t_hbm, bnd_hbm, out_hbm,
              bnd_vm, sidx_v, gidx_v, xval_v, vsel_v, scat0, scat1,
              rows0, rows1, acc, gsem0, gsem1, ssem0, ssem1, msem):
    c = lax.axis_index("c")
    s = lax.axis_index("s")

    _zero_acc(rows0, acc, s)

    pltpu.sync_copy(bnd_hbm, bnd_vm)
    bv = bnd_vm[...]
    is0 = c == 0
    lo_c = jnp.where(is0, bv[0], bv[2])
    hi_c = jnp.where(is0, bv[1], bv[3])
    mbase = jnp.where(is0, bv[4], bv[5])
    n_c = hi_c - lo_c
    p0 = lo_c + ((n_c * s) // NSUB // 8) * 8
    p1 = jnp.where(s == NSUB - 1, hi_c,
                   lo_c + ((n_c * (s + 1)) // NSUB // 8) * 8)
    nsup = (p1 - p0 + (KSUP - 1)) // KSUP

    plsc.subcore_barrier()

    lane = lax.iota(jnp.int32, L)
    rows = (rows0, rows1)
    scat = (scat0, scat1)
    gsem = (gsem0, gsem1)
    ssem = (ssem0, ssem1)

    def sup(i, _):
        base = pl.multiple_of(p0 + i * KSUP, 8)
        m1 = pltpu.async_copy(sidx_hbm.at[pl.ds(base, KSUP)], sidx_v,
                              msem)
        m2 = pltpu.async_copy(gidx_hbm.at[pl.ds(base, KSUP)], gidx_v,
                              msem)
        m3 = pltpu.async_copy(xval_hbm.at[pl.ds(base, KSUP)], xval_v,
                              msem)
        m1.wait()
        m2.wait()
        m3.wait()
        desc_s = [[], []]
        pltpu.async_copy(t_hbm.at[gidx_v.at[pl.ds(0, KX)]], rows[0],
                         gsem[0])
        for j in range(SUP):
            b = j % NBUF
            nb = (j + 1) % NBUF
            pltpu.make_async_copy(
                t_hbm.at[gidx_v.at[pl.ds(j * KX, KX)]], rows[b],
                gsem[b]).wait()
            for g in range(KX // L):
                sl_src = pl.ds(j * KX + g * L, L)
                sl_dst = pl.ds((g % (KH2 // L)) * L, L)
                d = sidx_v[sl_src]
                v = xval_v[sl_src]
                pos = base + j * KX + g * L + lane
                ok = (pos < p1) & (d >= mbase) & (d < mbase + NUM_WORDS)
                vsel_v[pl.ds(g * L, L)] = jnp.where(ok, v,
                                                    jnp.float32(0.0))
                scat[b][g // (KH2 // L), sl_dst] = (
                    jnp.clip(d - mbase, 0, NUM_WORDS - 1))
            if j + 1 < SUP:
                for dd in desc_s[nb]:
                    dd.wait()
                desc_s[nb] = []
                pltpu.async_copy(
                    t_hbm.at[gidx_v.at[pl.ds((j + 1) * KX, KX)]],
                    rows[nb], gsem[nb])
            desc_s[b] = []
            for hh in range(2):
                _scale_rows(rows[b], vsel_v, hh * KH2)
                desc_s[b].append(pltpu.async_copy(
                    rows[b].at[pl.ds(hh * KH2, KH2)],
                    acc.at[scat[b].at[hh]], ssem[b], add=True))
        for b in range(NBUF):
            for dd in desc_s[b]:
                dd.wait()
        return 0

    lax.fori_loop(0, nsup, sup, 0)

    plsc.subcore_barrier()
    row0 = s * RPT
    dst = out_hbm.at[pl.ds(c * NUM_WORDS, NUM_WORDS)]
    pltpu.sync_copy(acc.at[pl.ds(row0, RPT)], dst.at[pl.ds(row0, RPT)])

    @pl.when(s == NSUB - 1)
    def _():
        pltpu.sync_copy(acc.at[pl.ds(NSUB * RPT, 16)],
                        dst.at[pl.ds(NSUB * RPT, 16)])


_gss_call = functools.partial(
    pl.kernel,
    _gss_body,
    out_type=jax.ShapeDtypeStruct((NCORE * NUM_WORDS, HIDDEN), jnp.float32),
    mesh=_mesh,
    compiler_params=pltpu.CompilerParams(use_tc_tiling_on_sc=False),
    scratch_types=[
        pltpu.VMEM((L,), jnp.int32),
        pltpu.VMEM((KSUP,), jnp.int32),
        pltpu.VMEM((KSUP,), jnp.int32),
        pltpu.VMEM((KSUP,), jnp.float32),
        pltpu.VMEM((KX,), jnp.float32),
        pltpu.VMEM((2, KH2), jnp.int32),
        pltpu.VMEM((2, KH2), jnp.int32),
        pltpu.VMEM((KX, HIDDEN), jnp.float32),
        pltpu.VMEM((KX, HIDDEN), jnp.float32),
        pltpu.VMEM_SHARED((NUM_WORDS, HIDDEN), jnp.float32),
        pltpu.SemaphoreType.DMA,
        pltpu.SemaphoreType.DMA,
        pltpu.SemaphoreType.DMA,
        pltpu.SemaphoreType.DMA,
        pltpu.SemaphoreType.DMA,
    ],
)()


_EDGE_BOUNDS = (0, N_EDGES // NCORE, N_EDGES // NCORE, N_EDGES, 0, 0)


def _bounds_arr(vals):
    b = jnp.zeros((L,), jnp.int32)
    for i, v in enumerate(vals):
        b = b.at[i].set(v)
    return b


_MBLK = 1000


def _mm_relu_body(p_ref, w_ref, o_ref):
    x = p_ref[0] + p_ref[1]
    y = lax.dot_general(x, w_ref[...], (((1,), (1,)), ((), ())),
                        preferred_element_type=jnp.float32)
    o_ref[...] = jnp.maximum(y, 0.0)


def _mm_relu(p, w):
    return pl.pallas_call(
        _mm_relu_body,
        grid=(NUM_WORDS // _MBLK,),
        in_specs=[
            pl.BlockSpec((NCORE, _MBLK, HIDDEN), lambda i: (0, i, 0)),
            pl.BlockSpec((HIDDEN, HIDDEN), lambda i: (0, 0)),
        ],
        out_specs=pl.BlockSpec((_MBLK, HIDDEN), lambda i: (i, 0)),
        out_shape=jax.ShapeDtypeStruct((NUM_WORDS, HIDDEN), jnp.float32),
    )(p.reshape(NCORE, NUM_WORDS, HIDDEN), w)


def _mm3_ln_body(p_ref, w_ref, emb_ref, g_ref, b_ref, o_ref):
    x = p_ref[0] + p_ref[1]
    y = lax.dot_general(x, w_ref[...], (((1,), (1,)), ((), ())),
                        preferred_element_type=jnp.float32)
    y = jnp.maximum(y, 0.0)
    e = emb_ref[...]
    hh = jnp.float32(0.3) * e + jnp.float32(0.7) * y
    mean = jnp.mean(hh, axis=1, keepdims=True)
    var = jnp.mean((hh - mean) ** 2, axis=1, keepdims=True)
    wh = (hh - mean) / jnp.sqrt(var + jnp.float32(1e-5))
    wh = wh * g_ref[...] + b_ref[...]
    o_ref[...] = wh + e


def _mm3_ln(p, w, emb, gamma, beta):
    return pl.pallas_call(
        _mm3_ln_body,
        grid=(NUM_WORDS // _MBLK,),
        in_specs=[
            pl.BlockSpec((NCORE, _MBLK, HIDDEN), lambda i: (0, i, 0)),
            pl.BlockSpec((HIDDEN, HIDDEN), lambda i: (0, 0)),
            pl.BlockSpec((_MBLK, HIDDEN), lambda i: (i, 0)),
            pl.BlockSpec((1, HIDDEN), lambda i: (0, 0)),
            pl.BlockSpec((1, HIDDEN), lambda i: (0, 0)),
        ],
        out_specs=pl.BlockSpec((_MBLK, HIDDEN), lambda i: (i, 0)),
        out_shape=jax.ShapeDtypeStruct((NUM_WORDS, HIDDEN), jnp.float32),
    )(p.reshape(NCORE, NUM_WORDS, HIDDEN), w, emb, gamma.reshape(1, HIDDEN),
      beta.reshape(1, HIDDEN))


def _mlp_body(dh_ref, wm1_ref, bm1_ref, wm2_ref, bm2_ref, wc_ref, bc_ref,
              o_ref):
    x = dh_ref[...]
    h1 = lax.dot_general(x, wm1_ref[...], (((1,), (1,)), ((), ())),
                         preferred_element_type=jnp.float32)
    h1 = jnp.maximum(h1 + bm1_ref[...], 0.0)
    h2 = lax.dot_general(h1, wm2_ref[...], (((1,), (1,)), ((), ())),
                         preferred_element_type=jnp.float32)
    h2 = jnp.maximum(h2 + bm2_ref[...], 0.0)
    lg = lax.dot_general(h2, wc_ref[...], (((1,), (1,)), ((), ())),
                         preferred_element_type=jnp.float32)
    o_ref[...] = lg + bc_ref[...]


def _mlp(doc_h, wm1, bm1, wm2, bm2, wc, bc):
    hid2 = HIDDEN // 2
    wc_pad = jnp.zeros((8, hid2), jnp.float32).at[:2].set(wc)
    bc_pad = jnp.zeros((1, 8), jnp.float32).at[0, :2].set(bc)
    out = pl.pallas_call(
        _mlp_body,
        grid=(N_DOCS // _MBLK,),
        in_specs=[
            pl.BlockSpec((_MBLK, HIDDEN), lambda i: (i, 0)),
            pl.BlockSpec((HIDDEN, HIDDEN), lambda i: (0, 0)),
            pl.BlockSpec((1, HIDDEN), lambda i: (0, 0)),
            pl.BlockSpec((hid2, HIDDEN), lambda i: (0, 0)),
            pl.BlockSpec((1, hid2), lambda i: (0, 0)),
            pl.BlockSpec((8, hid2), lambda i: (0, 0)),
            pl.BlockSpec((1, 8), lambda i: (0, 0)),
        ],
        out_specs=pl.BlockSpec((_MBLK, 8), lambda i: (i, 0)),
        out_shape=jax.ShapeDtypeStruct((N_DOCS, 8), jnp.float32),
    )(doc_h, wm1, bm1.reshape(1, HIDDEN), wm2, bm2.reshape(1, hid2),
      wc_pad, bc_pad)
    return lax.slice(out, (0, 0), (N_DOCS, 2))


def kernel(A_indices, A_values, X_doc_idx, X_word_idx, X_values, emb,
           W1, W2, W3, ln_gamma, ln_beta, Wm1, bm1, Wm2, bm2, Wc, bc):
    epad = NNZ_PAD - N_EDGES
    row = jnp.pad(A_indices[0].astype(jnp.int32), (0, epad))
    col = jnp.pad(A_indices[1].astype(jnp.int32), (0, epad))
    aval = jnp.pad(A_values, (0, epad))
    ebnd = _bounds_arr(_EDGE_BOUNDS)

    p = _gss_call(row, col, aval, emb, ebnd)
    h = _mm_relu(p, W1)
    p = _gss_call(row, col, aval, h, ebnd)
    h = _mm_relu(p, W2)
    p = _gss_call(row, col, aval, h, ebnd)
    t = _mm3_ln(p, W3, emb, ln_gamma, ln_beta)

    didx = X_doc_idx.astype(jnp.int32)
    widx = X_word_idx.astype(jnp.int32)
    xpad = NNZ_PAD - NNZ_X
    didx_p = jnp.pad(didx, (0, xpad), constant_values=N_DOCS - 1)
    widx_p = jnp.pad(widx, (0, xpad))
    xval_p = jnp.pad(X_values, (0, xpad))

    split = jnp.searchsorted(didx, N_DOCS // NCORE).astype(jnp.int32)
    hi0 = ((split + 7) // 8) * 8
    lo1 = (split // 8) * 8
    xbnd = _bounds_arr((0, hi0, lo1, NNZ_X, 0, NUM_WORDS))

    doc_h = _gss_call(didx_p, widx_p, xval_p, t, xbnd)
    return _mlp(doc_h, Wm1, bm1, Wm2, bm2, Wc, bc)
